# baseline (speedup 1.0000x reference)
import jax
import jax.numpy as jnp
from jax import lax
from jax.experimental import pallas as pl
from jax.experimental.pallas import tpu as pltpu
from jax.experimental.pallas import tpu_sc as plsc

N_NODE = 10000
E = 160000
BASIS_UNITS = 4
MSG_RED = 64
MSG = 192
MSGH = 96
MSG_FULL = 384
OUT_UNITS = 256

NC = 2
NS = 16
NW = NC * NS
EPC = E // NC
EPW = E // NW
CHUNK = 128
NFULL = EPW // CHUNK
TAIL = EPW - NFULL * CHUNK
ROWS_PER_S = N_NODE // NS
GCH = 80
NGCH = 128
N_PAD = NGCH * GCH

def _sc_mesh():
    return plsc.VectorSubcoreMesh(core_axis_name="c", subcore_axis_name="s",
                                  num_cores=NC, num_subcores=NS)


def _wfull_body(att_ref, basis_ref, out_ref):
    a = att_ref[...]
    b = basis_ref[...]
    for r in range(2):
        acc = a[r, 0] * b[0]
        for k in range(1, BASIS_UNITS):
            acc = acc + a[r, k] * b[k]
        out_ref[r] = acc.astype(jnp.bfloat16)


def _wfull(att, basis):
    nblk = 10
    blk = N_NODE // nblk
    return pl.pallas_call(
        _wfull_body,
        grid=(nblk,),
        in_specs=[
            pl.BlockSpec((2, BASIS_UNITS), lambda i: (0, 0)),
            pl.BlockSpec((BASIS_UNITS, blk, MSG_RED), lambda i: (0, i, 0)),
        ],
        out_specs=pl.BlockSpec((2, blk, MSG_RED), lambda i: (0, i, 0)),
        out_shape=jax.ShapeDtypeStruct((2, N_NODE, MSG_RED), jnp.bfloat16),
    )(att, basis)


def _ctab_body(degd_ref, degi_ref, cd_ref, ci_ref):
    dd = degd_ref[0] + degd_ref[1]
    di = degi_ref[0] + degi_ref[1]
    cd_ref[...] = lax.rsqrt(jnp.maximum(dd, 1.0))
    ci_ref[...] = lax.rsqrt(jnp.maximum(di, 1.0))


def _ctab(degd_p, degi_p):
    nblk = 10
    blk = N_NODE // nblk
    deg_spec = pl.BlockSpec((NC, blk, 8), lambda i: (0, i, 0))
    c_spec = pl.BlockSpec((blk, 8), lambda i: (i, 0))
    return pl.pallas_call(
        _ctab_body,
        grid=(nblk,),
        in_specs=[deg_spec, deg_spec],
        out_specs=[c_spec, c_spec],
        out_shape=(jax.ShapeDtypeStruct((N_PAD, 8), jnp.float32),
                   jax.ShapeDtypeStruct((N_PAD, 8), jnp.float32)),
    )(degd_p, degi_p)


def _deg_body(s1m, s1t, d1m, d1t, s2m, s2t, d2m, d2t, ones8, zeros8,
              degd_p, degi_p,
              dacc, iacc, idxbig, idx_v8, ones_v, sem):
    c = lax.axis_index("c")
    s = lax.axis_index("s")
    w = c * NS + s
    row0 = s * ROWS_PER_S

    pltpu.sync_copy(zeros8, dacc.at[pl.ds(row0, ROWS_PER_S)])
    pltpu.sync_copy(zeros8, iacc.at[pl.ds(row0, ROWS_PER_S)])
    pltpu.sync_copy(ones8, ones_v)
    plsc.subcore_barrier()

    for sm, st, acc in ((s1m, s1t, dacc), (s2m, s2t, dacc),
                        (d1m, d1t, iacc), (d2m, d2t, iacc)):
        pltpu.sync_copy(sm.at[w], idxbig)
        def deg_chunk(j, acc=acc):
            pltpu.sync_copy(ones_v, acc.at[idxbig.at[j]], add=True)
        lax.fori_loop(0, NFULL, lambda j, _, f=deg_chunk: (f(j), 0)[1], 0)
        pltpu.sync_copy(st.at[w], idx_v8)
        pltpu.sync_copy(ones_v.at[pl.ds(0, TAIL)], acc.at[idx_v8], add=True)

    plsc.subcore_barrier()
    pltpu.sync_copy(dacc.at[pl.ds(row0, ROWS_PER_S)], degd_p.at[c, s])
    pltpu.sync_copy(iacc.at[pl.ds(row0, ROWS_PER_S)], degi_p.at[c, s])
    del sem


def _deg(edges_split):
    ones8 = jnp.ones((CHUNK, 8), jnp.float32)
    zeros8 = jnp.zeros((ROWS_PER_S, 8), jnp.float32)
    out_type = (
        jax.ShapeDtypeStruct((NC, NS, ROWS_PER_S, 8), jnp.float32),
        jax.ShapeDtypeStruct((NC, NS, ROWS_PER_S, 8), jnp.float32),
    )
    scratch = [
        pltpu.VMEM_SHARED((N_NODE, 8), jnp.float32),
        pltpu.VMEM_SHARED((N_NODE, 8), jnp.float32),
        pltpu.VMEM((NFULL, CHUNK), jnp.int32),
        pltpu.VMEM((TAIL,), jnp.int32),
        pltpu.VMEM((CHUNK, 8), jnp.float32),
        pltpu.SemaphoreType.DMA,
    ]
    f = pl.kernel(_deg_body, out_type=out_type, mesh=_sc_mesh(),
                  scratch_types=scratch,
                  compiler_params=pltpu.CompilerParams(use_tc_tiling_on_sc=False))
    return f(*edges_split, ones8, zeros8)


def _fbuild_body(dc0, dc1, dc2, ic0, ic1, ic2, w0, w1, cd8, ci8,
                 f1d, f2d, f1i, f2i,
                 gbufa, gbufb, fbuf, cbuf, idx4, sema, semb):
    c = lax.axis_index("c")
    s = lax.axis_index("s")
    w = c * NS + s

    wtabs = (w0, w1)
    bufs, sems = (gbufa, gbufb), (sema, semb)
    pend = []
    q = 0
    lanes16 = lax.iota(jnp.int32, 16)

    def drain():
        buf, sem, fouts, r, k, t = pend.pop()
        pltpu.make_async_copy(wtabs[0].at[idx4.at[0]], buf, sem).wait()
        def row(i, _, t=t):
            cv = plsc.load_gather(
                cbuf, [lanes16 * 0 + (t * GCH + i), lanes16 * 0])
            cv2 = plsc.pack(cv, cv, format=plsc.PackFormat.INTERLEAVED)
            for j in range(2):
                x = buf[i, pl.ds(j * 32, 32)]
                fbuf[i, pl.ds(j * 32, 32)] = x * cv2
            return 0
        lax.fori_loop(0, GCH, row, 0)
        nb = pl.multiple_of((w + NW * t) * GCH, 8)
        pltpu.sync_copy(
            fbuf, fouts[r].at[pl.ds(nb, GCH), pl.ds(k * MSG_RED, MSG_RED)])

    for side, cols, ctab, fouts in ((0, (dc0, dc1, dc2), cd8, (f1d, f2d)),
                                    (1, (ic0, ic1, ic2), ci8, (f1i, f2i))):
        for k in range(3):
            if pend:
                drain()
            pltpu.sync_copy(cols[k].at[:, w], idx4)
            for t in range(4):
                nb = pl.multiple_of((w + NW * t) * GCH, 8)
                pltpu.sync_copy(ctab.at[pl.ds(nb, GCH)],
                                cbuf.at[pl.ds(t * GCH, GCH)])
            for r in range(2):
                for t in range(4):
                    b = q % 2
                    pltpu.async_copy(wtabs[r].at[idx4.at[t]], bufs[b],
                                     sems[b])
                    if pend:
                        drain()
                    pend.append((bufs[b], sems[b], fouts, r, k, t))
                    q += 1
    drain()


def _fbuild(dcols, icols, w0, w1, cd8, ci8):
    out_type = tuple(
        jax.ShapeDtypeStruct((N_PAD, MSG), jnp.bfloat16) for _ in range(4))
    scratch = [
        pltpu.VMEM((GCH, MSG_RED), jnp.bfloat16),
        pltpu.VMEM((GCH, MSG_RED), jnp.bfloat16),
        pltpu.VMEM((GCH, MSG_RED), jnp.bfloat16),
        pltpu.VMEM((4 * GCH, 8), jnp.float32),
        pltpu.VMEM((4, GCH), jnp.int32),
        pltpu.SemaphoreType.DMA,
        pltpu.SemaphoreType.DMA,
    ]
    f = pl.kernel(_fbuild_body, out_type=out_type, mesh=_sc_mesh(),
                  scratch_types=scratch,
                  compiler_params=pltpu.CompilerParams(
                      use_tc_tiling_on_sc=False, needs_layout_passes=False))
    return f(dcols[0], dcols[1], dcols[2], icols[0], icols[1], icols[2],
             w0, w1, cd8, ci8)


def _conv_body(f1d, f2d, f1i, f2i,
               s1m, s1t, d1m, d1t, s2m, s2t, d2m, d2t, zeros125,
               rst_p, acc, sidx_all, didx_all, sidx8, didx8,
               msga, msgb, msg8, zbuf, sema, semb):
    c = lax.axis_index("c")
    s = lax.axis_index("s")
    w = c * NS + s
    row0 = s * ROWS_PER_S

    pltpu.sync_copy(zeros125, zbuf)

    convs = ((f1d, s1m, s1t, d1m, d1t),
             (f2d, s2m, s2t, d2m, d2t),
             (f1i, d1m, d1t, s1m, s1t),
             (f2i, d2m, d2t, s2m, s2t))
    for ci, (ftab, sm, st, dm, dt) in enumerate(convs):
        pltpu.sync_copy(sm.at[w], sidx_all)
        pltpu.sync_copy(dm.at[w], didx_all)
        pltpu.sync_copy(st.at[w], sidx8)
        pltpu.sync_copy(dt.at[w], didx8)
        for j in range(5):
            pltpu.sync_copy(zbuf, acc.at[pl.ds(row0 + j * 125, 125)])
        plsc.subcore_barrier()

        def start_g(j, buf, sem, ftab=ftab):
            pltpu.async_copy(ftab.at[sidx_all.at[j]], buf, sem)

        def wait_g(buf, sem, ftab=ftab):
            pltpu.make_async_copy(ftab.at[sidx_all.at[0]], buf, sem).wait()

        def scat(j, buf):
            pltpu.sync_copy(buf, acc.at[didx_all.at[j]], add=True)

        start_g(0, msga, sema)

        def pair(j2, _):
            ja = 2 * j2
            wait_g(msga, sema)
            start_g(ja + 1, msgb, semb)
            scat(ja, msga)
            wait_g(msgb, semb)
            start_g(ja + 2, msga, sema)
            scat(ja + 1, msgb)
            return 0
        lax.fori_loop(0, (NFULL - 1) // 2, pair, 0)
        wait_g(msga, sema)
        scat(NFULL - 1, msga)
        pltpu.async_copy(ftab.at[sidx8], msg8, sema).wait()
        pltpu.sync_copy(msg8, acc.at[didx8], add=True)

        plsc.subcore_barrier()
        pltpu.sync_copy(acc.at[pl.ds(row0, ROWS_PER_S)],
                        rst_p.at[ci, c, s])
        plsc.subcore_barrier()


def _conv(ftabs, edges_split):
    zeros125 = jnp.zeros((125, MSG), jnp.bfloat16)
    out_type = jax.ShapeDtypeStruct((4, NC, NS, ROWS_PER_S, MSG),
                                    jnp.bfloat16)
    scratch = [
        pltpu.VMEM_SHARED((N_NODE, MSG), jnp.bfloat16),
        pltpu.VMEM((NFULL, CHUNK), jnp.int32),
        pltpu.VMEM((NFULL, CHUNK), jnp.int32),
        pltpu.VMEM((TAIL,), jnp.int32),
        pltpu.VMEM((TAIL,), jnp.int32),
        pltpu.VMEM((CHUNK, MSG), jnp.bfloat16),
        pltpu.VMEM((CHUNK, MSG), jnp.bfloat16),
        pltpu.VMEM((TAIL, MSG), jnp.bfloat16),
        pltpu.VMEM((125, MSG), jnp.bfloat16),
        pltpu.SemaphoreType.DMA,
        pltpu.SemaphoreType.DMA,
    ]
    f = pl.kernel(_conv_body, out_type=out_type, mesh=_sc_mesh(),
                  scratch_types=scratch,
                  compiler_params=pltpu.CompilerParams(use_tc_tiling_on_sc=False))
    return f(*ftabs, *edges_split, zeros125)


def _proj_body(pa_ref, pb_ref, c_ref, w1_ref, w2_ref, b_ref, out_ref):
    cc = c_ref[:, 0:1]

    def m(ref):
        return (ref[0, 0].astype(jnp.float32)
                + ref[0, 1].astype(jnp.float32))
    h1 = m(pa_ref) * cc
    h2 = m(pb_ref) * cc
    out_ref[...] = (
        jnp.dot(h1, w1_ref[...], preferred_element_type=jnp.float32)
        + jnp.dot(h2, w2_ref[...], preferred_element_type=jnp.float32)
        + b_ref[...])


def _proj(rst_p, ca, cb, c_side, w1, w2, b2d):
    nblk = 10
    blk = N_NODE // nblk

    def p_spec(ci):
        return pl.BlockSpec((1, NC, blk, MSG),
                            lambda i, ci=ci: (ci, 0, i, 0))
    return pl.pallas_call(
        _proj_body,
        grid=(nblk,),
        in_specs=[
            p_spec(ca), p_spec(cb),
            pl.BlockSpec((blk, 8), lambda i: (i, 0)),
            pl.BlockSpec((MSG, OUT_UNITS), lambda i: (0, 0)),
            pl.BlockSpec((MSG, OUT_UNITS), lambda i: (0, 0)),
            pl.BlockSpec((1, OUT_UNITS), lambda i: (0, 0)),
        ],
        out_specs=pl.BlockSpec((blk, OUT_UNITS), lambda i: (i, 0)),
        out_shape=jax.ShapeDtypeStruct((N_NODE, OUT_UNITS), jnp.float32),
    )(rst_p, rst_p, c_side, w1, w2, b2d)


def kernel(drug_feat, dis_feat, edge_index_r1, edge_index_r2,
           att, basis, W_ufc, b_ufc):
    def split(e):
        m = e.astype(jnp.int32).reshape(NW, EPW)
        return (m[:, :NFULL * CHUNK].reshape(NW, NFULL, CHUNK),
                m[:, NFULL * CHUNK:])

    edges_split = []
    for arr in (edge_index_r1, edge_index_r2):
        for row in (0, 1):
            edges_split.extend(split(arr[row]))

    def colpack(feat, k):
        col = jnp.pad(feat[:, k].astype(jnp.int32), (0, N_PAD - N_NODE))
        return col.reshape(4, NW, GCH)
    dcols = [colpack(drug_feat, k) for k in range(3)]
    icols = [colpack(dis_feat, k) for k in range(3)]

    wf = _wfull(att, basis)
    degd_p, degi_p = _deg(edges_split)
    degd_p = degd_p.reshape(NC, N_NODE, 8)
    degi_p = degi_p.reshape(NC, N_NODE, 8)
    cd8, ci8 = _ctab(degd_p, degi_p)
    ftabs = _fbuild(dcols, icols, wf[0], wf[1], cd8, ci8)
    rst_p = _conv(ftabs, edges_split)
    rst_p = rst_p.reshape(4, NC, N_NODE, MSG)

    w1 = W_ufc[:MSG]
    w2 = W_ufc[MSG:]
    b2d = b_ufc[None, :]
    out_drug = _proj(rst_p, 2, 3, cd8, w1, w2, b2d)
    out_dis = _proj(rst_p, 0, 1, ci8, w1, w2, b2d)
    return jnp.concatenate([out_drug, out_dis], axis=0)
